# baseline (speedup 1.0000x reference)
import functools

import jax
import jax.numpy as jnp
from jax import lax
from jax.experimental import pallas as pl
from jax.experimental.pallas import tpu as pltpu
from jax.experimental.pallas import tpu_sc as plsc

NC = 2
NS = 16
NW = NC * NS

N_ATOMS = 100000
D = 512
N0 = 4096
A = 24
TOTAL = N0 * A
PER_TILE = TOTAL // NW
CHUNK = 128
NCHUNK = PER_TILE // CHUNK
ZCH = 6256
TBL = NS * ZCH

SCALE = 1.0 / float(TOTAL)

_sc_mesh = plsc.VectorSubcoreMesh(core_axis_name="c", subcore_axis_name="s")


@functools.partial(
    pl.kernel,
    out_type=jax.ShapeDtypeStruct((NC, TBL), jnp.float32),
    mesh=_sc_mesh,
    scratch_types=[
        pltpu.VMEM((NCHUNK, CHUNK), jnp.int32),
        pltpu.VMEM((CHUNK,), jnp.float32),
        pltpu.VMEM((ZCH,), jnp.float32),
        pltpu.VMEM_SHARED((TBL,), jnp.float32),
    ],
)
def _histogram(idx_hbm, out_hbm, idx_v, ones_v, zeros_v, table_sh):
    cid = lax.axis_index("c")
    sid = lax.axis_index("s")
    w = cid * NS + sid

    ones16 = jnp.full((16,), 1.0, jnp.float32)
    for i in range(CHUNK // 16):
        ones_v[pl.ds(i * 16, 16)] = ones16

    zero16 = jnp.zeros((16,), jnp.float32)

    def _zbody(i, carry):
        zeros_v[pl.ds(i * 16, 16)] = zero16
        return carry

    lax.fori_loop(0, ZCH // 16, _zbody, 0)

    pltpu.sync_copy(idx_hbm.at[w], idx_v)

    pltpu.sync_copy(zeros_v, table_sh.at[pl.ds(sid * ZCH, ZCH)])
    plsc.subcore_barrier()

    for j in range(NCHUNK):
        pltpu.sync_copy(ones_v, table_sh.at[idx_v.at[j]], add=True)
    plsc.subcore_barrier()

    @pl.when(sid == 0)
    def _():
        pltpu.sync_copy(table_sh, out_hbm.at[cid])


BK = 10000
NKB = N_ATOMS // BK


def _mv_body(c_ref, x_ref, o_ref):
    k = pl.program_id(0)

    @pl.when(k == 0)
    def _():
        o_ref[...] = jnp.zeros_like(o_ref)

    c = c_ref[0, pl.ds(k, 1), :] + c_ref[1, pl.ds(k, 1), :]
    o_ref[...] += jnp.dot(c, x_ref[...], preferred_element_type=jnp.float32)

    @pl.when(k == NKB - 1)
    def _():
        o_ref[...] = o_ref[...] * SCALE


_matvec = pl.pallas_call(
    _mv_body,
    grid=(NKB,),
    in_specs=[
        pl.BlockSpec((NC, NKB, BK), lambda k: (0, 0, 0)),
        pl.BlockSpec((BK, D), lambda k: (k, 0)),
    ],
    out_specs=pl.BlockSpec((1, D), lambda k: (0, 0)),
    out_shape=jax.ShapeDtypeStruct((1, D), jnp.float32),
)


def kernel(x_atom_fea, node_atom_idx):
    idx = node_atom_idx.astype(jnp.int32).reshape(NW, NCHUNK, CHUNK)
    counts = _histogram(idx)
    counts3 = counts[:, :N_ATOMS].reshape(NC, NKB, BK)
    return _matvec(counts3, x_atom_fea)


# speedup vs baseline: 1.0359x; 1.0359x over previous
import functools

import jax
import jax.numpy as jnp
from jax import lax
from jax.experimental import pallas as pl
from jax.experimental.pallas import tpu as pltpu
from jax.experimental.pallas import tpu_sc as plsc

NC = 2
NS = 16
NW = NC * NS

N_ATOMS = 100000
D = 512
N0 = 4096
A = 24
TOTAL = N0 * A
PER_TILE = TOTAL // NW
CHUNK = 128
NCHUNK = PER_TILE // CHUNK
ZCH = 6256
TBL = NS * ZCH

SCALE = 1.0 / float(TOTAL)

_sc_mesh = plsc.VectorSubcoreMesh(core_axis_name="c", subcore_axis_name="s")


@functools.partial(
    pl.kernel,
    out_type=jax.ShapeDtypeStruct((NC, TBL), jnp.float32),
    mesh=_sc_mesh,
    scratch_types=[
        pltpu.VMEM((NCHUNK, CHUNK), jnp.int32),
        pltpu.VMEM((CHUNK,), jnp.float32),
        pltpu.VMEM((ZCH,), jnp.float32),
        pltpu.VMEM_SHARED((TBL,), jnp.float32),
    ],
)
def _histogram(idx_hbm, out_hbm, idx_v, ones_v, zeros_v, table_sh):
    cid = lax.axis_index("c")
    sid = lax.axis_index("s")
    w = cid * NS + sid

    ones16 = jnp.full((16,), 1.0, jnp.float32)
    for i in range(CHUNK // 16):
        ones_v[pl.ds(i * 16, 16)] = ones16

    zero16 = jnp.zeros((16,), jnp.float32)

    def _zbody(i, carry):
        zeros_v[pl.ds(i * 16, 16)] = zero16
        return carry

    lax.fori_loop(0, ZCH // 16, _zbody, 0)

    pltpu.sync_copy(idx_hbm.at[w], idx_v)

    pltpu.sync_copy(zeros_v, table_sh.at[pl.ds(sid * ZCH, ZCH)])
    plsc.subcore_barrier()

    for j in range(NCHUNK):
        pltpu.sync_copy(ones_v, table_sh.at[idx_v.at[j]], add=True)
    plsc.subcore_barrier()

    @pl.when(sid == 0)
    def _():
        pltpu.sync_copy(table_sh, out_hbm.at[cid])


BK = 5000
NKB = N_ATOMS // BK


def _mv_body(c_ref, x_ref, o_ref):
    k = pl.program_id(0)

    @pl.when(k == 0)
    def _():
        o_ref[...] = jnp.zeros_like(o_ref)

    c = c_ref[0, pl.ds(k, 1), :] + c_ref[1, pl.ds(k, 1), :]
    o_ref[...] += jnp.dot(c, x_ref[...], preferred_element_type=jnp.float32)

    @pl.when(k == NKB - 1)
    def _():
        o_ref[...] = o_ref[...] * SCALE


_matvec = pl.pallas_call(
    _mv_body,
    grid=(NKB,),
    in_specs=[
        pl.BlockSpec((NC, NKB, BK), lambda k: (0, 0, 0)),
        pl.BlockSpec((BK, D), lambda k: (k, 0)),
    ],
    out_specs=pl.BlockSpec((1, D), lambda k: (0, 0)),
    out_shape=jax.ShapeDtypeStruct((1, D), jnp.float32),
)


def kernel(x_atom_fea, node_atom_idx):
    idx = node_atom_idx.astype(jnp.int32).reshape(NW, NCHUNK, CHUNK)
    counts = _histogram(idx)
    counts3 = counts[:, :N_ATOMS].reshape(NC, NKB, BK)
    return _matvec(counts3, x_atom_fea)


# speedup vs baseline: 1.0852x; 1.0476x over previous
import functools

import jax
import jax.numpy as jnp
from jax import lax
from jax.experimental import pallas as pl
from jax.experimental.pallas import tpu as pltpu
from jax.experimental.pallas import tpu_sc as plsc

NC = 2
NS = 16
NW = NC * NS

N_ATOMS = 100000
D = 512
N0 = 4096
A = 24
TOTAL = N0 * A
PER_TILE = TOTAL // NW
CHUNK = 128
NCHUNK = PER_TILE // CHUNK
ZCH = 6256
TBL = NS * ZCH

SCALE = 1.0 / float(TOTAL)

_sc_mesh = plsc.VectorSubcoreMesh(core_axis_name="c", subcore_axis_name="s")


@functools.partial(
    pl.kernel,
    out_type=jax.ShapeDtypeStruct((NC, TBL), jnp.float32),
    mesh=_sc_mesh,
    scratch_types=[
        pltpu.VMEM((NCHUNK, CHUNK), jnp.int32),
        pltpu.VMEM((CHUNK,), jnp.float32),
        pltpu.VMEM((ZCH,), jnp.float32),
        pltpu.VMEM_SHARED((TBL,), jnp.float32),
        pltpu.SemaphoreType.DMA,
        pltpu.SemaphoreType.DMA,
    ],
)
def _histogram(idx_hbm, out_hbm, idx_v, ones_v, zeros_v, table_sh,
               sem_idx, sem_sc):
    cid = lax.axis_index("c")
    sid = lax.axis_index("s")
    w = cid * NS + sid

    idx_cp = pltpu.async_copy(idx_hbm.at[w], idx_v, sem_idx)

    ones16 = jnp.full((16,), 1.0, jnp.float32)
    for i in range(CHUNK // 16):
        ones_v[pl.ds(i * 16, 16)] = ones16

    zero16 = jnp.zeros((16,), jnp.float32)

    def _zbody(i, carry):
        for u in range(17):
            zeros_v[pl.ds((i * 17 + u) * 16, 16)] = zero16
        return carry

    lax.fori_loop(0, ZCH // (16 * 17), _zbody, 0)

    pltpu.sync_copy(zeros_v, table_sh.at[pl.ds(sid * ZCH, ZCH)])
    idx_cp.wait()
    plsc.subcore_barrier()

    cps = [
        pltpu.async_copy(ones_v, table_sh.at[idx_v.at[j]], sem_sc, add=True)
        for j in range(NCHUNK)
    ]
    for cp in cps:
        cp.wait()
    plsc.subcore_barrier()

    @pl.when(sid == 0)
    def _():
        pltpu.sync_copy(table_sh, out_hbm.at[cid])


BK = 4000
NKB = N_ATOMS // BK


def _mv_body(c_ref, x_ref, o_ref):
    k = pl.program_id(0)

    @pl.when(k == 0)
    def _():
        o_ref[...] = jnp.zeros_like(o_ref)

    c = c_ref[0, pl.ds(k, 1), :] + c_ref[1, pl.ds(k, 1), :]
    o_ref[...] += jnp.dot(c, x_ref[...], preferred_element_type=jnp.float32)

    @pl.when(k == NKB - 1)
    def _():
        o_ref[...] = o_ref[...] * SCALE


_matvec = pl.pallas_call(
    _mv_body,
    grid=(NKB,),
    in_specs=[
        pl.BlockSpec((NC, NKB, BK), lambda k: (0, 0, 0)),
        pl.BlockSpec((BK, D), lambda k: (k, 0)),
    ],
    out_specs=pl.BlockSpec((1, D), lambda k: (0, 0)),
    out_shape=jax.ShapeDtypeStruct((1, D), jnp.float32),
)


def kernel(x_atom_fea, node_atom_idx):
    idx = node_atom_idx.astype(jnp.int32).reshape(NW, NCHUNK, CHUNK)
    counts = _histogram(idx)
    counts3 = counts[:, :N_ATOMS].reshape(NC, NKB, BK)
    return _matvec(counts3, x_atom_fea)
